# Initial kernel scaffold; baseline (speedup 1.0000x reference)
import jax
import jax.numpy as jnp
from jax.experimental import pallas as pl


def kernel(local, lidx0, lidx1, lidx2, lidx3):
    raise NotImplementedError("write your pallas kernel here")


# speedup vs baseline: 3.3718x; 3.3718x over previous
import functools

import jax
import jax.numpy as jnp
from jax import lax
from jax.experimental import pallas as pl
from jax.experimental.pallas import tpu as pltpu
from jax.experimental.pallas import tpu_sc as plsc

N = 100000
D = 256
HALO = 12500
OUT_ROWS = N + 3 * HALO
NC, NS = 2, 16
NW = NC * NS
CHUNK = 128
NFULL = OUT_ROWS // CHUNK
TAIL_BASE = OUT_ROWS - CHUNK
ITERS = -(-NFULL // NW)

_mesh = plsc.VectorSubcoreMesh(core_axis_name="c", subcore_axis_name="s")


@functools.partial(
    pl.kernel,
    out_type=jax.ShapeDtypeStruct((OUT_ROWS, D), jnp.float32),
    mesh=_mesh,
    scratch_types=[
        pltpu.VMEM((CHUNK,), jnp.int32),
        pltpu.VMEM((CHUNK, D), jnp.float32),
        pltpu.VMEM((CHUNK,), jnp.int32),
        pltpu.SemaphoreType.DMA,
    ],
)
def _gather_rows(local_hbm, idx_hbm, tidx_hbm, oidx_hbm, out_hbm,
                 idx_v, buf_v, oidx_v, sem):
    wid = lax.axis_index("s") * NC + lax.axis_index("c")

    def body(i, carry):
        j = wid + i * NW

        @pl.when(j < NFULL)
        def _():
            base = j * CHUNK
            pltpu.sync_copy(idx_hbm.at[pl.ds(base, CHUNK)], idx_v)
            pltpu.async_copy(local_hbm.at[idx_v], buf_v, sem).wait()
            pltpu.sync_copy(buf_v, out_hbm.at[pl.ds(base, CHUNK)])

        return carry

    lax.fori_loop(0, ITERS, body, 0)

    @pl.when(wid == NW - 1)
    def _():
        pltpu.sync_copy(tidx_hbm, idx_v)
        pltpu.sync_copy(oidx_hbm, oidx_v)
        pltpu.async_copy(local_hbm.at[idx_v], buf_v, sem).wait()
        pltpu.async_copy(buf_v, out_hbm.at[oidx_v], sem).wait()


def kernel(local, lidx0, lidx1, lidx2, lidx3):
    del lidx0
    idx = jnp.concatenate([
        jnp.arange(N, dtype=jnp.int32),
        lidx1.astype(jnp.int32),
        lidx2.astype(jnp.int32),
        lidx3.astype(jnp.int32),
    ])
    tidx = lax.slice(idx, (TAIL_BASE,), (OUT_ROWS,))
    oidx = jnp.arange(TAIL_BASE, OUT_ROWS, dtype=jnp.int32)
    return _gather_rows(local, idx, tidx, oidx)


# speedup vs baseline: 4.3642x; 1.2943x over previous
import functools

import jax
import jax.numpy as jnp
from jax import lax
from jax.experimental import pallas as pl
from jax.experimental.pallas import tpu as pltpu
from jax.experimental.pallas import tpu_sc as plsc

N = 100000
D = 256
HALO = 12500
OUT_ROWS = N + 3 * HALO
NC, NS = 2, 16
NW = NC * NS
CHUNK = 128
NFULL = OUT_ROWS // CHUNK
TAIL_BASE = OUT_ROWS - CHUNK
ITERS = -(-NFULL // NW)

_mesh = plsc.VectorSubcoreMesh(core_axis_name="c", subcore_axis_name="s")


@functools.partial(
    pl.kernel,
    out_type=jax.ShapeDtypeStruct((OUT_ROWS, D), jnp.float32),
    mesh=_mesh,
    scratch_types=[
        pltpu.VMEM((CHUNK,), jnp.int32),
        pltpu.VMEM((CHUNK,), jnp.int32),
        pltpu.VMEM((CHUNK, D), jnp.float32),
        pltpu.VMEM((CHUNK, D), jnp.float32),
        pltpu.VMEM((CHUNK,), jnp.int32),
        pltpu.SemaphoreType.DMA,
        pltpu.SemaphoreType.DMA,
        pltpu.SemaphoreType.DMA,
        pltpu.SemaphoreType.DMA,
        pltpu.SemaphoreType.DMA,
        pltpu.SemaphoreType.DMA,
    ],
)
def _gather_rows(local_hbm, idx_hbm, tidx_hbm, oidx_hbm, out_hbm,
                 idx0_v, idx1_v, buf0_v, buf1_v, oidx_v,
                 semg0, semg1, semw0, semw1, semi0, semi1):
    wid = lax.axis_index("s") * NC + lax.axis_index("c")
    idx_v = (idx0_v, idx1_v)
    buf_v = (buf0_v, buf1_v)
    semg = (semg0, semg1)
    semw = (semw0, semw1)
    semi = (semi0, semi1)

    def chunk_of(i):
        return wid + i * NW

    def valid(i):
        return chunk_of(i) < NFULL

    def start_gather(b):
        pltpu.async_copy(local_hbm.at[idx_v[b]], buf_v[b], semg[b])

    def wait_gather(b):
        pltpu.make_async_copy(
            local_hbm.at[pl.ds(0, CHUNK)], buf_v[b], semg[b]).wait()

    def start_write(b, j):
        pltpu.async_copy(buf_v[b], out_hbm.at[pl.ds(j * CHUNK, CHUNK)],
                         semw[b])

    def wait_write(b):
        pltpu.make_async_copy(
            buf_v[b], out_hbm.at[pl.ds(0, CHUNK)], semw[b]).wait()

    def start_idx(b, j):
        pltpu.async_copy(idx_hbm.at[pl.ds(j * CHUNK, CHUNK)], idx_v[b],
                         semi[b])

    def wait_idx(b):
        pltpu.make_async_copy(
            idx_hbm.at[pl.ds(0, CHUNK)], idx_v[b], semi[b]).wait()

    start_idx(0, chunk_of(0))
    wait_idx(0)
    start_gather(0)
    start_idx(1, chunk_of(1))

    def body(k, carry):
        for b in (0, 1):
            i = k * 2 + b
            o = 1 - b

            @pl.when((i >= 1) & valid(i - 1))
            def _():
                wait_write(o)

            @pl.when(valid(i + 1))
            def _():
                wait_idx(o)
                start_gather(o)

            @pl.when(valid(i))
            def _():
                wait_gather(b)
                start_write(b, chunk_of(i))

            @pl.when(valid(i + 2))
            def _():
                start_idx(b, chunk_of(i + 2))

        return carry

    lax.fori_loop(0, ITERS // 2, body, 0)

    @pl.when(valid(ITERS - 1))
    def _():
        wait_write((ITERS - 1) % 2)

    @pl.when(wid == NW - 1)
    def _():
        pltpu.sync_copy(tidx_hbm, idx0_v)
        pltpu.sync_copy(oidx_hbm, oidx_v)
        pltpu.async_copy(local_hbm.at[idx0_v], buf0_v, semg0).wait()
        pltpu.async_copy(buf0_v, out_hbm.at[oidx_v], semw0).wait()


def kernel(local, lidx0, lidx1, lidx2, lidx3):
    del lidx0
    idx = jnp.concatenate([
        jnp.arange(N, dtype=jnp.int32),
        lidx1.astype(jnp.int32),
        lidx2.astype(jnp.int32),
        lidx3.astype(jnp.int32),
    ])
    tidx = lax.slice(idx, (TAIL_BASE,), (OUT_ROWS,))
    oidx = jnp.arange(TAIL_BASE, OUT_ROWS, dtype=jnp.int32)
    return _gather_rows(local, idx, tidx, oidx)
